# Initial kernel scaffold; baseline (speedup 1.0000x reference)
import jax
import jax.numpy as jnp
from jax.experimental import pallas as pl


def kernel(feat_rows, feat_cols, feat_values, adj_row, adj_col, adj_values, weight):
    raise NotImplementedError("write your pallas kernel here")


# speedup vs baseline: 1.0900x; 1.0900x over previous
import functools

import jax
import jax.numpy as jnp
from jax import lax
from jax.experimental import pallas as pl
from jax.experimental.pallas import tpu as pltpu
from jax.experimental.pallas import tpu_sc as plsc

N = 10000
D = 128
O = 128
NNZ = 320000
L = 16
NC = 2
NS = 16
NW = NC * NS
CPT = O // NW
CH = 1600
NCHUNK = NNZ // CH
NGRP = CH // L


def _body(fr, fc, fv, ar, ac, av, w_hbm, out_hbm,
          wvm, xw, ob, rb, cb, vb, sem):
  wid = lax.axis_index("s") * NC + lax.axis_index("c")
  colbase = wid * CPT

  pltpu.sync_copy(w_hbm, wvm)

  def zero(i, carry):
    sl = pl.ds(i * L, L)
    xw[sl] = jnp.zeros((L,), jnp.float32)
    ob[sl] = jnp.zeros((L,), jnp.float32)
    return carry
  lax.fori_loop(0, N * CPT // L, zero, 0)

  def spmm_pass(rows_hbm, cols_hbm, vals_hbm, gather_ref, gmul, acc_ref):
    goff = jnp.where(gmul == D, colbase, 0).astype(jnp.int32)

    def chunk(k, carry):
      base = pl.ds(k * CH, CH)
      pltpu.sync_copy(rows_hbm.at[base], rb)
      pltpu.sync_copy(cols_hbm.at[base], cb)
      pltpu.sync_copy(vals_hbm.at[base], vb)

      def grp(g, carry2):
        sl = pl.ds(g * L, L)
        r = rb[sl]
        c = cb[sl]
        v = vb[sl]
        gidx = c * gmul + goff
        sidx = r * CPT
        for j in range(CPT):
          wrow = plsc.load_gather(gather_ref, [gidx + j])
          plsc.addupdate_scatter(acc_ref, [sidx + j], v * wrow)
        return carry2
      lax.fori_loop(0, NGRP, grp, 0)
      return carry
    lax.fori_loop(0, NCHUNK, chunk, 0)

  spmm_pass(fr, fc, fv, wvm, jnp.int32(D), xw)
  spmm_pass(ar, ac, av, xw, jnp.int32(CPT), ob)

  def relu(i, carry):
    sl = pl.ds(i * L, L)
    ob[sl] = jnp.maximum(ob[sl], 0.0)
    return carry
  lax.fori_loop(0, N * CPT // L, relu, 0)
  pltpu.sync_copy(ob, out_hbm.at[wid])


@functools.partial(jax.jit)
def _sc_call(fr, fc, fv, ar, ac, av, wflat):
  mesh = plsc.VectorSubcoreMesh(core_axis_name="c", subcore_axis_name="s")
  f = pl.kernel(
      _body,
      out_type=jax.ShapeDtypeStruct((NW, N * CPT), jnp.float32),
      mesh=mesh,
      scratch_types=[
          pltpu.VMEM((D * O,), jnp.float32),
          pltpu.VMEM((N * CPT,), jnp.float32),
          pltpu.VMEM((N * CPT,), jnp.float32),
          pltpu.VMEM((CH,), jnp.int32),
          pltpu.VMEM((CH,), jnp.int32),
          pltpu.VMEM((CH,), jnp.float32),
          pltpu.SemaphoreType.DMA,
      ],
      compiler_params=pltpu.CompilerParams(needs_layout_passes=False),
  )
  return f(fr, fc, fv, ar, ac, av, wflat)


def kernel(feat_rows, feat_cols, feat_values, adj_row, adj_col, adj_values,
           weight):
  blocks = _sc_call(feat_rows, feat_cols, feat_values,
                    adj_row, adj_col, adj_values, weight.reshape(-1))
  return blocks.reshape(NW, N, CPT).transpose(1, 0, 2).reshape(N, O)


# speedup vs baseline: 2.7023x; 2.4793x over previous
import functools

import jax
import jax.numpy as jnp
from jax import lax
from jax.experimental import pallas as pl
from jax.experimental.pallas import tpu as pltpu
from jax.experimental.pallas import tpu_sc as plsc

N = 10000
D = 128
O = 128
NNZ = 320000
L = 16
NC = 2
NS = 16
NW = NC * NS
CPT = O // NW
CH = 3200
NCHUNK = NNZ // CH
NGRP = CH // L
UNROLL = 4


def _body(fr, fc, fv, ar, ac, av, w_hbm, out_hbm,
          wvm, xw, ob, rb0, cb0, vb0, rb1, cb1, vb1, sem0, sem1, wsem):
  wid = lax.axis_index("s") * NC + lax.axis_index("c")
  colbase = wid * CPT

  wcp = pltpu.async_copy(w_hbm, wvm, wsem)

  @plsc.parallel_loop(0, N * CPT // L, unroll=UNROLL)
  def _zero(i):
    sl = pl.ds(i * L, L)
    xw[sl] = jnp.zeros((L,), jnp.float32)
    ob[sl] = jnp.zeros((L,), jnp.float32)

  wcp.wait()

  def spmm_pass(rows_hbm, cols_hbm, vals_hbm, gather_ref, gmul, goff, acc_ref):
    bufs = ((rb0, cb0, vb0, sem0), (rb1, cb1, vb1, sem1))

    def start(k, b):
      rbuf, cbuf, vbuf, sem = bufs[b]
      sl = pl.ds(k * CH, CH)
      pltpu.async_copy(rows_hbm.at[sl], rbuf, sem)
      pltpu.async_copy(cols_hbm.at[sl], cbuf, sem)
      pltpu.async_copy(vals_hbm.at[sl], vbuf, sem)

    def drain(b):
      rbuf, cbuf, vbuf, sem = bufs[b]
      pltpu.make_async_copy(rows_hbm.at[pl.ds(0, CH)], rbuf, sem).wait()
      pltpu.make_async_copy(cols_hbm.at[pl.ds(0, CH)], cbuf, sem).wait()
      pltpu.make_async_copy(vals_hbm.at[pl.ds(0, CH)], vbuf, sem).wait()

    def process(b):
      rbuf, cbuf, vbuf, _ = bufs[b]

      @plsc.parallel_loop(0, NGRP, unroll=UNROLL)
      def _grp(g):
        sl = pl.ds(g * L, L)
        r = rbuf[sl]
        c = cbuf[sl]
        v = vbuf[sl]
        gidx = c * gmul + goff
        sidx = r * CPT
        for j in range(CPT):
          wrow = plsc.load_gather(gather_ref, [gidx + j])
          plsc.addupdate_scatter(acc_ref, [sidx + j], v * wrow)

    start(0, 0)
    start(1, 1)

    def step(k2, carry):
      k = k2 * 2
      drain(0)
      process(0)

      @pl.when(k + 2 < NCHUNK)
      def _():
        start(k + 2, 0)

      drain(1)
      process(1)

      @pl.when(k + 3 < NCHUNK)
      def _():
        start(k + 3, 1)
      return carry
    lax.fori_loop(0, NCHUNK // 2, step, 0)

  spmm_pass(fr, fc, fv, wvm, jnp.int32(D), colbase.astype(jnp.int32), xw)
  spmm_pass(ar, ac, av, xw, jnp.int32(CPT), jnp.int32(0), ob)

  @plsc.parallel_loop(0, N * CPT // L, unroll=UNROLL)
  def _relu(i):
    sl = pl.ds(i * L, L)
    ob[sl] = jnp.maximum(ob[sl], 0.0)

  pltpu.sync_copy(ob, out_hbm.at[wid])


@functools.partial(jax.jit)
def _sc_call(fr, fc, fv, ar, ac, av, wflat):
  mesh = plsc.VectorSubcoreMesh(core_axis_name="c", subcore_axis_name="s")
  f = pl.kernel(
      _body,
      out_type=jax.ShapeDtypeStruct((NW, N * CPT), jnp.float32),
      mesh=mesh,
      scratch_types=[
          pltpu.VMEM((D * O,), jnp.float32),
          pltpu.VMEM((N * CPT,), jnp.float32),
          pltpu.VMEM((N * CPT,), jnp.float32),
          pltpu.VMEM((CH,), jnp.int32),
          pltpu.VMEM((CH,), jnp.int32),
          pltpu.VMEM((CH,), jnp.float32),
          pltpu.VMEM((CH,), jnp.int32),
          pltpu.VMEM((CH,), jnp.int32),
          pltpu.VMEM((CH,), jnp.float32),
          pltpu.SemaphoreType.DMA,
          pltpu.SemaphoreType.DMA,
          pltpu.SemaphoreType.DMA,
      ],
      compiler_params=pltpu.CompilerParams(needs_layout_passes=False),
  )
  return f(fr, fc, fv, ar, ac, av, wflat)


def kernel(feat_rows, feat_cols, feat_values, adj_row, adj_col, adj_values,
           weight):
  blocks = _sc_call(feat_rows, feat_cols, feat_values,
                    adj_row, adj_col, adj_values, weight.reshape(-1))
  return blocks.reshape(NW, N, CPT).transpose(1, 0, 2).reshape(N, O)
